# baseline (device time: 19334 ns/iter reference)
import jax
import jax.numpy as jnp
from jax import lax
from jax.experimental import pallas as pl
from jax.experimental.pallas import tpu as pltpu

N_DEV = 4
SUB = 48


def kernel(A, B):
    m, k_dim = A.shape
    _, n = B.shape
    m2, m4, m8 = m // 2, m // 4, m // 8

    def body(a_hbm, b_hbm, out_ref, a_ref, b_ref, pbfa, pbfb, s2a, s2b,
             r1a, r1b, r2a, r2b, copy_sems, ssem, rsem):
        me = lax.axis_index("i")
        bit0 = me & 1
        bit1 = (me >> 1) & 1
        ka = bit0 ^ bit1
        kb = bit1
        pa1 = me ^ 1
        pa2 = me ^ 3
        ha_row = ka * m4
        hb_row = m2 + kb * m4

        cp_b = pltpu.make_async_copy(b_hbm, b_ref, copy_sems.at[0])
        cp_b.start()
        cp_a = pltpu.make_async_copy(a_hbm, a_ref, copy_sems.at[1])
        cp_a.start()
        cp_b.wait()
        b = b_ref[:, :].astype(jnp.bfloat16)
        cp_a.wait()

        def xchg(src, dst, sem_i, partner):
            rd = pltpu.make_async_remote_copy(
                src_ref=src, dst_ref=dst,
                send_sem=ssem.at[sem_i], recv_sem=rsem.at[sem_i],
                device_id=(partner,),
                device_id_type=pl.DeviceIdType.MESH,
            )
            rd.start()
            return rd

        def dot96(start):
            a_c = a_ref[pl.ds(start, m8), :].astype(jnp.bfloat16)
            return jnp.dot(a_c, b, preferred_element_type=jnp.float32)

        pa_row = (1 - ka) * m4
        pb_row = m2 + (1 - kb) * m4
        pbfa[pl.ds(0, m8), :] = dot96(pa_row).astype(jnp.bfloat16)
        pbfb[pl.ds(0, m8), :] = dot96(pb_row).astype(jnp.bfloat16)

        barrier_sem = pltpu.get_barrier_semaphore()
        for nbr in (pa1, pa2):
            pl.semaphore_signal(
                barrier_sem, inc=1,
                device_id=(nbr,), device_id_type=pl.DeviceIdType.MESH,
            )
        pl.semaphore_wait(barrier_sem, 2)

        def sub(ref, c):
            return ref.at[pl.ds(c * SUB, SUB), :]

        x1a, x1b = {}, {}
        for c in (0, 1):
            x1a[c] = xchg(sub(pbfa, c), sub(r1a, c), c, pa1)
            x1b[c] = xchg(sub(pbfb, c), sub(r1b, c), 4 + c, pa2)
        pbfa[pl.ds(m8, m8), :] = dot96(pa_row + m8).astype(jnp.bfloat16)
        for c in (2, 3):
            x1a[c] = xchg(sub(pbfa, c), sub(r1a, c), c, pa1)
        pbfb[pl.ds(m8, m8), :] = dot96(pb_row + m8).astype(jnp.bfloat16)
        for c in (2, 3):
            x1b[c] = xchg(sub(pbfb, c), sub(r1b, c), 4 + c, pa2)

        va = [dot96(ha_row).astype(jnp.bfloat16),
              dot96(ha_row + m8).astype(jnp.bfloat16)]
        vb = [dot96(hb_row).astype(jnp.bfloat16),
              dot96(hb_row + m8).astype(jnp.bfloat16)]

        def own(v, c):
            return v[c // 2][(c % 2) * SUB:(c % 2 + 1) * SUB, :]

        x2a, x2b = {}, {}
        for c in range(4):
            x1a[c].wait_recv()
            s2a[pl.ds(c * SUB, SUB), :] = own(va, c) + r1a[pl.ds(c * SUB, SUB), :]
            x2a[c] = xchg(sub(s2a, c), sub(r2a, c), 8 + c, pa2)
            x1b[c].wait_recv()
            s2b[pl.ds(c * SUB, SUB), :] = own(vb, c) + r1b[pl.ds(c * SUB, SUB), :]
            x2b[c] = xchg(sub(s2b, c), sub(r2b, c), 12 + c, pa1)

        x3a, x3b = {}, {}
        for c in range(4):
            ra = pl.ds(ha_row + c * SUB, SUB)
            x2a[c].wait_recv()
            out_ref[ra, :] = (
                s2a[pl.ds(c * SUB, SUB), :] + r2a[pl.ds(c * SUB, SUB), :]
            )
            x3a[c] = xchg(out_ref.at[ra, :], out_ref.at[ra, :], 16 + c, pa1)
            rb = pl.ds(hb_row + c * SUB, SUB)
            x2b[c].wait_recv()
            out_ref[rb, :] = (
                s2b[pl.ds(c * SUB, SUB), :] + r2b[pl.ds(c * SUB, SUB), :]
            )
            x3b[c] = xchg(out_ref.at[rb, :], out_ref.at[rb, :], 20 + c, pa2)

        for c in range(4):
            x3a[c].wait_recv()
            x3b[c].wait_recv()

        for grp in (x1a, x1b, x2a, x2b, x3a, x3b):
            for rd in grp.values():
                rd.wait_send()

    bf = jnp.bfloat16
    return pl.pallas_call(
        body,
        out_shape=jax.ShapeDtypeStruct((m, n), bf),
        in_specs=[
            pl.BlockSpec(memory_space=pl.ANY),
            pl.BlockSpec(memory_space=pl.ANY),
        ],
        out_specs=pl.BlockSpec(memory_space=pltpu.VMEM),
        scratch_shapes=[
            pltpu.VMEM((m, k_dim), jnp.float32),
            pltpu.VMEM((k_dim, n), jnp.float32),
            pltpu.VMEM((m4, n), bf),
            pltpu.VMEM((m4, n), bf),
            pltpu.VMEM((m4, n), bf),
            pltpu.VMEM((m4, n), bf),
            pltpu.VMEM((m4, n), bf),
            pltpu.VMEM((m4, n), bf),
            pltpu.VMEM((m4, n), bf),
            pltpu.VMEM((m4, n), bf),
            pltpu.SemaphoreType.DMA((2,)),
            pltpu.SemaphoreType.DMA((24,)),
            pltpu.SemaphoreType.DMA((24,)),
        ],
        compiler_params=pltpu.CompilerParams(collective_id=0),
    )(A, B)
